# baseline (speedup 1.0000x reference)
import jax
import jax.numpy as jnp
from jax import lax
from jax.experimental import pallas as pl
from jax.experimental.pallas import tpu as pltpu
from jax.experimental.pallas import tpu_sc as plsc

_BATCH = 4096
_NUM_CORES = 2
_NUM_SUBCORES = 16
_NUM_WORKERS = _NUM_CORES * _NUM_SUBCORES
_BPW = _BATCH // _NUM_WORKERS
_LANES = 16

_D_BETAS = 16
_D_BODY_POSE = 156
_D_ORIENT = 3
_D_TRANSL = 3


def _sc_lookup(frame_ids, betas_w, body_pose_w, global_orient_w, transl_w,
               out_betas, out_bp, out_go, out_tr,
               idx_v, zidx_v, betas_v, bp_v, go_v, tr_v, sem):
    wid = lax.axis_index("s") * _NUM_CORES + lax.axis_index("c")
    base = wid * _BPW

    zeros = jnp.zeros((_LANES,), jnp.int32)
    for i in range(_BPW // _LANES):
        zidx_v[pl.ds(i * _LANES, _LANES)] = zeros

    pltpu.sync_copy(frame_ids.at[pl.ds(base, _BPW)], idx_v)

    c_bp = pltpu.make_async_copy(body_pose_w.at[idx_v], bp_v, sem)
    c_go = pltpu.make_async_copy(global_orient_w.at[idx_v], go_v, sem)
    c_tr = pltpu.make_async_copy(transl_w.at[idx_v], tr_v, sem)
    c_be = pltpu.make_async_copy(betas_w.at[zidx_v], betas_v, sem)
    c_bp.start()
    c_go.start()
    c_tr.start()
    c_be.start()
    c_bp.wait()
    c_go.wait()
    c_tr.wait()
    c_be.wait()

    pltpu.sync_copy(bp_v, out_bp.at[pl.ds(base, _BPW)])
    pltpu.sync_copy(go_v, out_go.at[pl.ds(base, _BPW)])
    pltpu.sync_copy(tr_v, out_tr.at[pl.ds(base, _BPW)])
    pltpu.sync_copy(betas_v, out_betas.at[pl.ds(base, _BPW)])


@jax.jit
def _run(frame_ids, betas_w, body_pose_w, global_orient_w, transl_w):
    call = pl.kernel(
        _sc_lookup,
        out_type=(
            jax.ShapeDtypeStruct((_BATCH, _D_BETAS), jnp.float32),
            jax.ShapeDtypeStruct((_BATCH, _D_BODY_POSE), jnp.float32),
            jax.ShapeDtypeStruct((_BATCH, _D_ORIENT), jnp.float32),
            jax.ShapeDtypeStruct((_BATCH, _D_TRANSL), jnp.float32),
        ),
        mesh=plsc.VectorSubcoreMesh(
            core_axis_name="c", subcore_axis_name="s",
            num_cores=_NUM_CORES, num_subcores=_NUM_SUBCORES,
        ),
        scratch_types=[
            pltpu.VMEM((_BPW,), jnp.int32),
            pltpu.VMEM((_BPW,), jnp.int32),
            pltpu.VMEM((_BPW, _D_BETAS), jnp.float32),
            pltpu.VMEM((_BPW, _D_BODY_POSE), jnp.float32),
            pltpu.VMEM((_BPW, _D_ORIENT), jnp.float32),
            pltpu.VMEM((_BPW, _D_TRANSL), jnp.float32),
            pltpu.SemaphoreType.DMA,
        ],
        compiler_params=pltpu.CompilerParams(use_tc_tiling_on_sc=False),
    )
    return call(frame_ids, betas_w, body_pose_w, global_orient_w, transl_w)


def kernel(frame_ids, betas_w, body_pose_w, global_orient_w, transl_w):
    return _run(frame_ids, betas_w, body_pose_w, global_orient_w, transl_w)


# speedup vs baseline: 3.9359x; 3.9359x over previous
import jax
import jax.numpy as jnp
from jax import lax
from jax.experimental import pallas as pl
from jax.experimental.pallas import tpu as pltpu
from jax.experimental.pallas import tpu_sc as plsc

_BATCH = 4096
_NUM_CORES = 2
_NUM_SUBCORES = 16
_NUM_WORKERS = _NUM_CORES * _NUM_SUBCORES
_BPW = _BATCH // _NUM_WORKERS

_D_BETAS = 16
_D_BODY_POSE = 156
_D_ORIENT = 3
_D_TRANSL = 3


def _sc_lookup(frame_ids, betas_w, body_pose_w, global_orient_w, transl_w,
               out_betas, out_bp, out_go, out_tr,
               idx_s, betas_v, bp_v, go_v, tr_v,
               sem_bp, sem_go, sem_tr, sem_x):
    wid = lax.axis_index("s") * _NUM_CORES + lax.axis_index("c")
    base = wid * _BPW

    pltpu.sync_copy(frame_ids.at[pl.ds(base, _BPW)], idx_s)

    def fire(g, carry):
        vec = idx_s[pl.ds(g * 16, 16)]
        for j in range(16):
            r = vec[j]
            i = g * 16 + j
            pltpu.make_async_copy(
                body_pose_w.at[pl.ds(r, 1)], bp_v.at[pl.ds(i, 1)], sem_bp).start()
            pltpu.make_async_copy(
                global_orient_w.at[pl.ds(r, 1)], go_v.at[pl.ds(i, 1)], sem_go).start()
            pltpu.make_async_copy(
                transl_w.at[pl.ds(r, 1)], tr_v.at[pl.ds(i, 1)], sem_tr).start()
        return carry

    lax.fori_loop(0, _BPW // 16, fire, 0)

    pltpu.sync_copy(betas_w.at[pl.ds(0, 1)], betas_v.at[pl.ds(0, 1)])
    brow = betas_v[0, :]
    for i in range(1, _BPW):
        betas_v[i, :] = brow

    pltpu.make_async_copy(body_pose_w.at[pl.ds(0, _BPW)], bp_v, sem_bp).wait()
    pltpu.make_async_copy(global_orient_w.at[pl.ds(0, _BPW)], go_v, sem_go).wait()
    pltpu.make_async_copy(transl_w.at[pl.ds(0, _BPW)], tr_v, sem_tr).wait()

    c0 = pltpu.make_async_copy(bp_v, out_bp.at[pl.ds(base, _BPW)], sem_x)
    c1 = pltpu.make_async_copy(go_v, out_go.at[pl.ds(base, _BPW)], sem_x)
    c2 = pltpu.make_async_copy(tr_v, out_tr.at[pl.ds(base, _BPW)], sem_x)
    c3 = pltpu.make_async_copy(betas_v, out_betas.at[pl.ds(base, _BPW)], sem_x)
    c0.start()
    c1.start()
    c2.start()
    c3.start()
    c0.wait()
    c1.wait()
    c2.wait()
    c3.wait()


@jax.jit
def _run(frame_ids, betas_w, body_pose_w, global_orient_w, transl_w):
    call = pl.kernel(
        _sc_lookup,
        out_type=(
            jax.ShapeDtypeStruct((_BATCH, _D_BETAS), jnp.float32),
            jax.ShapeDtypeStruct((_BATCH, _D_BODY_POSE), jnp.float32),
            jax.ShapeDtypeStruct((_BATCH, _D_ORIENT), jnp.float32),
            jax.ShapeDtypeStruct((_BATCH, _D_TRANSL), jnp.float32),
        ),
        mesh=plsc.VectorSubcoreMesh(
            core_axis_name="c", subcore_axis_name="s",
            num_cores=_NUM_CORES, num_subcores=_NUM_SUBCORES,
        ),
        scratch_types=[
            pltpu.VMEM((_BPW,), jnp.int32),
            pltpu.VMEM((_BPW, _D_BETAS), jnp.float32),
            pltpu.VMEM((_BPW, _D_BODY_POSE), jnp.float32),
            pltpu.VMEM((_BPW, _D_ORIENT), jnp.float32),
            pltpu.VMEM((_BPW, _D_TRANSL), jnp.float32),
            pltpu.SemaphoreType.DMA,
            pltpu.SemaphoreType.DMA,
            pltpu.SemaphoreType.DMA,
            pltpu.SemaphoreType.DMA,
        ],
        compiler_params=pltpu.CompilerParams(use_tc_tiling_on_sc=True),
    )
    return call(frame_ids, betas_w, body_pose_w, global_orient_w, transl_w)


def kernel(frame_ids, betas_w, body_pose_w, global_orient_w, transl_w):
    return _run(frame_ids, betas_w, body_pose_w, global_orient_w, transl_w)


# speedup vs baseline: 3.9583x; 1.0057x over previous
import jax
import jax.numpy as jnp
from jax import lax
from jax.experimental import pallas as pl
from jax.experimental.pallas import tpu as pltpu
from jax.experimental.pallas import tpu_sc as plsc

_BATCH = 4096
_NUM_CORES = 2
_NUM_SUBCORES = 16
_NUM_WORKERS = _NUM_CORES * _NUM_SUBCORES
_BPW = _BATCH // _NUM_WORKERS

_D_BETAS = 16
_D_BODY_POSE = 156
_D_ORIENT = 3
_D_TRANSL = 3


def _sc_lookup(frame_ids, betas_w, body_pose_w, global_orient_w, transl_w,
               out_betas, out_bp, out_go, out_tr,
               idx_s, betas_v, bp_v, go_v, tr_v,
               sem_bp, sem_go, sem_tr, sem_x):
    wid = lax.axis_index("s") * _NUM_CORES + lax.axis_index("c")
    base = wid * _BPW

    pltpu.sync_copy(frame_ids.at[pl.ds(base, _BPW)], idx_s)

    pltpu.make_async_copy(
        body_pose_w.at[idx_s, pl.ds(0, 128)], bp_v.at[:, pl.ds(0, 128)],
        sem_bp).start()

    def fire(g, carry):
        vec = idx_s[pl.ds(g * 16, 16)]
        for j in range(16):
            r = vec[j]
            i = g * 16 + j
            pltpu.make_async_copy(
                body_pose_w.at[pl.ds(r, 1), pl.ds(128, 28)],
                bp_v.at[pl.ds(i, 1), pl.ds(128, 28)], sem_bp).start()
            pltpu.make_async_copy(
                global_orient_w.at[pl.ds(r, 1)], go_v.at[pl.ds(i, 1)], sem_go).start()
            pltpu.make_async_copy(
                transl_w.at[pl.ds(r, 1)], tr_v.at[pl.ds(i, 1)], sem_tr).start()
        return carry

    lax.fori_loop(0, _BPW // 16, fire, 0)

    pltpu.sync_copy(betas_w.at[pl.ds(0, 1)], betas_v.at[pl.ds(0, 1)])
    brow = betas_v[0, :]
    for i in range(1, _BPW):
        betas_v[i, :] = brow

    pltpu.make_async_copy(body_pose_w.at[pl.ds(0, _BPW)], bp_v, sem_bp).wait()
    pltpu.make_async_copy(global_orient_w.at[pl.ds(0, _BPW)], go_v, sem_go).wait()
    pltpu.make_async_copy(transl_w.at[pl.ds(0, _BPW)], tr_v, sem_tr).wait()

    c0 = pltpu.make_async_copy(bp_v, out_bp.at[pl.ds(base, _BPW)], sem_x)
    c1 = pltpu.make_async_copy(go_v, out_go.at[pl.ds(base, _BPW)], sem_x)
    c2 = pltpu.make_async_copy(tr_v, out_tr.at[pl.ds(base, _BPW)], sem_x)
    c3 = pltpu.make_async_copy(betas_v, out_betas.at[pl.ds(base, _BPW)], sem_x)
    c0.start()
    c1.start()
    c2.start()
    c3.start()
    c0.wait()
    c1.wait()
    c2.wait()
    c3.wait()


@jax.jit
def _run(frame_ids, betas_w, body_pose_w, global_orient_w, transl_w):
    call = pl.kernel(
        _sc_lookup,
        out_type=(
            jax.ShapeDtypeStruct((_BATCH, _D_BETAS), jnp.float32),
            jax.ShapeDtypeStruct((_BATCH, _D_BODY_POSE), jnp.float32),
            jax.ShapeDtypeStruct((_BATCH, _D_ORIENT), jnp.float32),
            jax.ShapeDtypeStruct((_BATCH, _D_TRANSL), jnp.float32),
        ),
        mesh=plsc.VectorSubcoreMesh(
            core_axis_name="c", subcore_axis_name="s",
            num_cores=_NUM_CORES, num_subcores=_NUM_SUBCORES,
        ),
        scratch_types=[
            pltpu.VMEM((_BPW,), jnp.int32),
            pltpu.VMEM((_BPW, _D_BETAS), jnp.float32),
            pltpu.VMEM((_BPW, _D_BODY_POSE), jnp.float32),
            pltpu.VMEM((_BPW, _D_ORIENT), jnp.float32),
            pltpu.VMEM((_BPW, _D_TRANSL), jnp.float32),
            pltpu.SemaphoreType.DMA,
            pltpu.SemaphoreType.DMA,
            pltpu.SemaphoreType.DMA,
            pltpu.SemaphoreType.DMA,
        ],
        compiler_params=pltpu.CompilerParams(use_tc_tiling_on_sc=True),
    )
    return call(frame_ids, betas_w, body_pose_w, global_orient_w, transl_w)


def kernel(frame_ids, betas_w, body_pose_w, global_orient_w, transl_w):
    return _run(frame_ids, betas_w, body_pose_w, global_orient_w, transl_w)


# speedup vs baseline: 5.7281x; 1.4471x over previous
import jax
import jax.numpy as jnp
from jax import lax
from jax.experimental import pallas as pl
from jax.experimental.pallas import tpu as pltpu
from jax.experimental.pallas import tpu_sc as plsc

_BATCH = 4096
_NUM_CORES = 2
_NUM_SUBCORES = 16
_NUM_WORKERS = _NUM_CORES * _NUM_SUBCORES
_BPW = _BATCH // _NUM_WORKERS

_NUM_FRAMES = 100000
_D_BETAS = 16
_D_BODY_POSE = 156
_D_SMALL = 3
_LANE = 128


def _sc_lookup(frame_ids, betas_w, body_pose_w, go_flat, tr_flat,
               out_betas_t, out_bp, out_go_t, out_tr_t,
               idx_s, widx_v, brow_v, betas_v, bp_v, go_v, tr_v,
               sem_bp, sem_sm, sem_x):
    wid = lax.axis_index("s") * _NUM_CORES + lax.axis_index("c")
    base = wid * _BPW

    pltpu.sync_copy(frame_ids.at[pl.ds(base, _BPW)], idx_s)

    pltpu.make_async_copy(
        body_pose_w.at[idx_s, pl.ds(0, _LANE)], bp_v.at[:, pl.ds(0, _LANE)],
        sem_bp).start()

    for g in range(_BPW // 16):
        vec = idx_s[pl.ds(g * 16, 16)]
        for r in range(_D_SMALL):
            widx_v[r, pl.ds(g * 16, 16)] = vec + (r * _NUM_FRAMES)

    for r in range(_D_SMALL):
        pltpu.make_async_copy(
            go_flat.at[widx_v.at[r]], go_v.at[r], sem_sm).start()
        pltpu.make_async_copy(
            tr_flat.at[widx_v.at[r]], tr_v.at[r], sem_sm).start()

    def fire(g, carry):
        vec = idx_s[pl.ds(g * 16, 16)]
        for j in range(16):
            r = vec[j]
            i = g * 16 + j
            pltpu.make_async_copy(
                body_pose_w.at[pl.ds(r, 1), pl.ds(_LANE, _D_BODY_POSE - _LANE)],
                bp_v.at[pl.ds(i, 1), pl.ds(_LANE, _D_BODY_POSE - _LANE)],
                sem_bp).start()
        return carry

    lax.fori_loop(0, _BPW // 16, fire, 0)

    pltpu.sync_copy(betas_w, brow_v)
    brow = brow_v[0, :]
    for j in range(_D_BETAS):
        val = jnp.full((16,), brow[j], jnp.float32)
        for k in range(_BPW // 16):
            betas_v[j, pl.ds(k * 16, 16)] = val

    pltpu.make_async_copy(go_flat.at[pl.ds(0, _BPW)], go_v.at[0], sem_sm).wait()
    pltpu.make_async_copy(go_flat.at[pl.ds(0, _BPW)], go_v.at[1], sem_sm).wait()
    pltpu.make_async_copy(go_flat.at[pl.ds(0, _BPW)], go_v.at[2], sem_sm).wait()
    pltpu.make_async_copy(tr_flat.at[pl.ds(0, _BPW)], tr_v.at[0], sem_sm).wait()
    pltpu.make_async_copy(tr_flat.at[pl.ds(0, _BPW)], tr_v.at[1], sem_sm).wait()
    pltpu.make_async_copy(tr_flat.at[pl.ds(0, _BPW)], tr_v.at[2], sem_sm).wait()
    pltpu.make_async_copy(body_pose_w.at[pl.ds(0, _BPW)], bp_v, sem_bp).wait()

    lbase = pl.multiple_of(base, _LANE)
    c0 = pltpu.make_async_copy(bp_v, out_bp.at[pl.ds(base, _BPW)], sem_x)
    c1 = pltpu.make_async_copy(go_v, out_go_t.at[:, pl.ds(lbase, _BPW)], sem_x)
    c2 = pltpu.make_async_copy(tr_v, out_tr_t.at[:, pl.ds(lbase, _BPW)], sem_x)
    c3 = pltpu.make_async_copy(betas_v, out_betas_t.at[:, pl.ds(lbase, _BPW)], sem_x)
    c0.start()
    c1.start()
    c2.start()
    c3.start()
    c0.wait()
    c1.wait()
    c2.wait()
    c3.wait()


@jax.jit
def _run(frame_ids, betas_w, body_pose_w, global_orient_w, transl_w):
    call = pl.kernel(
        _sc_lookup,
        out_type=(
            jax.ShapeDtypeStruct((_D_BETAS, _BATCH), jnp.float32),
            jax.ShapeDtypeStruct((_BATCH, _D_BODY_POSE), jnp.float32),
            jax.ShapeDtypeStruct((_D_SMALL, _BATCH), jnp.float32),
            jax.ShapeDtypeStruct((_D_SMALL, _BATCH), jnp.float32),
        ),
        mesh=plsc.VectorSubcoreMesh(
            core_axis_name="c", subcore_axis_name="s",
            num_cores=_NUM_CORES, num_subcores=_NUM_SUBCORES,
        ),
        scratch_types=[
            pltpu.VMEM((_BPW,), jnp.int32),
            pltpu.VMEM((_D_SMALL, _BPW), jnp.int32),
            pltpu.VMEM((1, _D_BETAS), jnp.float32),
            pltpu.VMEM((_D_BETAS, _BPW), jnp.float32),
            pltpu.VMEM((_BPW, _D_BODY_POSE), jnp.float32),
            pltpu.VMEM((_D_SMALL, _BPW), jnp.float32),
            pltpu.VMEM((_D_SMALL, _BPW), jnp.float32),
            pltpu.SemaphoreType.DMA,
            pltpu.SemaphoreType.DMA,
            pltpu.SemaphoreType.DMA,
        ],
        compiler_params=pltpu.CompilerParams(
            use_tc_tiling_on_sc=True, needs_layout_passes=False),
    )
    betas_t, bp, go_res_t, tr_res_t = call(
        frame_ids, betas_w, body_pose_w,
        global_orient_w.T.reshape(-1), transl_w.T.reshape(-1))
    return betas_t.T, bp, go_res_t.T, tr_res_t.T


def kernel(frame_ids, betas_w, body_pose_w, global_orient_w, transl_w):
    return _run(frame_ids, betas_w, body_pose_w, global_orient_w, transl_w)


# speedup vs baseline: 6.9208x; 1.2082x over previous
import jax
import jax.numpy as jnp
from jax import lax
from jax.experimental import pallas as pl
from jax.experimental.pallas import tpu as pltpu
from jax.experimental.pallas import tpu_sc as plsc

_BATCH = 4096
_NUM_CORES = 2
_NUM_SUBCORES = 16
_NUM_WORKERS = _NUM_CORES * _NUM_SUBCORES
_BPW = _BATCH // _NUM_WORKERS

_NUM_FRAMES = 100000
_FPW = _NUM_FRAMES // _NUM_WORKERS
_D_BETAS = 16
_D_BP = 156
_D_SMALL = 3
_LANE = 128
_SLAB_W = 3328
_SLAB_W_LAST = 3200
_LAST_OFF = 96768
_TAIL_START = 99968
_CH = 256
_NSG = 20

def _sc_lookup(frame_ids, betas_w, bp_t, bp_tail, go_flat, tr_flat,
               out_betas_t, out_bp, out_go_t, out_tr_t,
               idx_all, widx_v, mc_v, mp_v, slab_v, ext_v,
               brow_v, betas_v, go_v, tr_v,
               sem_bp, sem_sm, sem_x):
    wid = lax.axis_index("s") * _NUM_CORES + lax.axis_index("c")
    base = wid * _BPW
    iota16 = jnp.arange(16, dtype=jnp.int32)

    pltpu.sync_copy(frame_ids, idx_all)

    for g in range(_BPW // 16):
        vec = idx_all[pl.ds(base + g * 16, 16)]
        for r in range(_D_SMALL):
            widx_v[r, pl.ds(g * 16, 16)] = vec + (r * _NUM_FRAMES)
    for r in range(_D_SMALL):
        pltpu.make_async_copy(
            go_flat.at[widx_v.at[r]], go_v.at[r], sem_sm).start()
        pltpu.make_async_copy(
            tr_flat.at[widx_v.at[r]], tr_v.at[r], sem_sm).start()

    lo = wid * _FPW
    hi = lo + _FPW
    slab_off = pl.multiple_of((lo // _LANE) * _LANE, _LANE)

    def scan(g, off):
        vec = idx_all[pl.ds(g * 16, 16)]
        m = jnp.logical_and(vec >= lo, vec < hi)
        plsc.store_compressed(mc_v.at[pl.ds(off, 16)], vec, mask=m)
        pos = g * 16 + iota16
        plsc.store_compressed(mp_v.at[pl.ds(off, 16)], pos, mask=m)
        return off + plsc.all_reduce_population_count(m)[0]

    num_match = lax.fori_loop(0, _BATCH // 16, scan, jnp.int32(0))

    pltpu.sync_copy(betas_w, brow_v)
    brow = brow_v[0, :]
    for j in range(_D_BETAS):
        val = jnp.full((16,), brow[j], jnp.float32)
        for k in range(_BPW // 16):
            betas_v[j, pl.ds(k * 16, 16)] = val

    def chunk_body(ch, carry):
        cbase = ch * _CH
        n_valid = jnp.minimum(_CH, num_match - cbase)
        n_mg = (n_valid + 15) // 16

        for s in range(_NSG):
            nrows = min(8, _D_BP - s * 8)

            @pl.when(wid < _NUM_WORKERS - 1)
            def _():
                pltpu.sync_copy(
                    bp_t.at[pl.ds(s * 8, nrows), pl.ds(slab_off, _SLAB_W)],
                    slab_v.at[pl.ds(0, nrows)])

            @pl.when(wid == _NUM_WORKERS - 1)
            def _():
                pltpu.sync_copy(
                    bp_t.at[pl.ds(s * 8, nrows), pl.ds(_LAST_OFF, _SLAB_W_LAST)],
                    slab_v.at[pl.ds(0, nrows), pl.ds(0, _SLAB_W_LAST)])

            def extract(mg, c2):
                mbase = cbase + mg * 16
                cvec = mc_v[pl.ds(mbase, 16)]
                valid = (mbase + iota16) < num_match
                sel = jnp.where(valid, cvec - slab_off, 0)
                erow = mg * 16 + iota16
                for r8 in range(nrows):
                    vals = plsc.load_gather(
                        slab_v, [jnp.full((16,), r8, jnp.int32), sel])
                    plsc.store_scatter(
                        ext_v, [erow, jnp.full((16,), s * 8 + r8, jnp.int32)],
                        vals, mask=valid)
                return c2

            lax.fori_loop(0, n_mg, extract, 0)

        def fire(mg, c2):
            pvec = mp_v[pl.ds(cbase + mg * 16, 16)]
            cvec = mc_v[pl.ds(cbase + mg * 16, 16)]
            for j in range(16):
                v = (cbase + mg * 16 + j) < num_match
                in_tail = cvec[j] >= _TAIL_START

                @pl.when(jnp.logical_and(v, jnp.logical_not(in_tail)))
                def _():
                    pltpu.make_async_copy(
                        ext_v.at[pl.ds(mg * 16 + j, 1)],
                        out_bp.at[pl.ds(pvec[j], 1)], sem_bp).start()

                @pl.when(jnp.logical_and(v, in_tail))
                def _():
                    pltpu.make_async_copy(
                        bp_tail.at[pl.ds(cvec[j] - _TAIL_START, 1)],
                        out_bp.at[pl.ds(pvec[j], 1)], sem_bp).start()
            return c2

        lax.fori_loop(0, n_mg, fire, 0)

        def drain(_, c2):
            pltpu.make_async_copy(
                ext_v.at[pl.ds(0, 1)], out_bp.at[pl.ds(0, 1)], sem_bp).wait()
            return c2

        lax.fori_loop(0, n_valid, drain, 0)
        return carry

    nch = (num_match + _CH - 1) // _CH
    lax.fori_loop(0, nch, chunk_body, 0)

    pltpu.make_async_copy(go_flat.at[pl.ds(0, _BPW)], go_v.at[0], sem_sm).wait()
    pltpu.make_async_copy(go_flat.at[pl.ds(0, _BPW)], go_v.at[1], sem_sm).wait()
    pltpu.make_async_copy(go_flat.at[pl.ds(0, _BPW)], go_v.at[2], sem_sm).wait()
    pltpu.make_async_copy(tr_flat.at[pl.ds(0, _BPW)], tr_v.at[0], sem_sm).wait()
    pltpu.make_async_copy(tr_flat.at[pl.ds(0, _BPW)], tr_v.at[1], sem_sm).wait()
    pltpu.make_async_copy(tr_flat.at[pl.ds(0, _BPW)], tr_v.at[2], sem_sm).wait()

    lbase = pl.multiple_of(base, _LANE)
    c1 = pltpu.make_async_copy(go_v, out_go_t.at[:, pl.ds(lbase, _BPW)], sem_x)
    c2 = pltpu.make_async_copy(tr_v, out_tr_t.at[:, pl.ds(lbase, _BPW)], sem_x)
    c3 = pltpu.make_async_copy(
        betas_v, out_betas_t.at[:, pl.ds(lbase, _BPW)], sem_x)
    c1.start()
    c2.start()
    c3.start()
    c1.wait()
    c2.wait()
    c3.wait()


@jax.jit
def _run(frame_ids, betas_w, body_pose_w, global_orient_w, transl_w):
    call = pl.kernel(
        _sc_lookup,
        out_type=(
            jax.ShapeDtypeStruct((_D_BETAS, _BATCH), jnp.float32),
            jax.ShapeDtypeStruct((_BATCH, _D_BP), jnp.float32),
            jax.ShapeDtypeStruct((_D_SMALL, _BATCH), jnp.float32),
            jax.ShapeDtypeStruct((_D_SMALL, _BATCH), jnp.float32),
        ),
        mesh=plsc.VectorSubcoreMesh(
            core_axis_name="c", subcore_axis_name="s",
            num_cores=_NUM_CORES, num_subcores=_NUM_SUBCORES,
        ),
        scratch_types=[
            pltpu.VMEM((_BATCH,), jnp.int32),
            pltpu.VMEM((_D_SMALL, _BPW), jnp.int32),
            pltpu.VMEM((_BATCH + 16,), jnp.int32),
            pltpu.VMEM((_BATCH + 16,), jnp.int32),
            pltpu.VMEM((8, _SLAB_W), jnp.float32),
            pltpu.VMEM((_CH, _D_BP), jnp.float32),
            pltpu.VMEM((1, _D_BETAS), jnp.float32),
            pltpu.VMEM((_D_BETAS, _BPW), jnp.float32),
            pltpu.VMEM((_D_SMALL, _BPW), jnp.float32),
            pltpu.VMEM((_D_SMALL, _BPW), jnp.float32),
            pltpu.SemaphoreType.DMA,
            pltpu.SemaphoreType.DMA,
            pltpu.SemaphoreType.DMA,
        ],
        compiler_params=pltpu.CompilerParams(
            use_tc_tiling_on_sc=True, needs_layout_passes=False),
    )
    betas_t, bp, go_res_t, tr_res_t = call(
        frame_ids, betas_w, body_pose_w.T, body_pose_w[_TAIL_START:],
        global_orient_w.T.reshape(-1), transl_w.T.reshape(-1))
    return betas_t.T, bp, go_res_t.T, tr_res_t.T


def kernel(frame_ids, betas_w, body_pose_w, global_orient_w, transl_w):
    return _run(frame_ids, betas_w, body_pose_w, global_orient_w, transl_w)


# speedup vs baseline: 8.3605x; 1.2080x over previous
import jax
import jax.numpy as jnp
from jax import lax
from jax.experimental import pallas as pl
from jax.experimental.pallas import tpu as pltpu
from jax.experimental.pallas import tpu_sc as plsc

_BATCH = 4096
_NUM_CORES = 2
_NUM_SUBCORES = 16
_NUM_WORKERS = _NUM_CORES * _NUM_SUBCORES
_BPW = _BATCH // _NUM_WORKERS

_NUM_FRAMES = 100000
_FPW = _NUM_FRAMES // _NUM_WORKERS
_D_BETAS = 16
_D_BP = 156
_D_SMALL = 3
_LANE = 128
_SLAB_W = 3328
_SLAB_W_LAST = 3200
_LAST_OFF = 96768
_TAIL_START = 99968
_CH = 192
_NSG = 20

def _sc_lookup(frame_ids, betas_w, bp_t, bp_tail, go_flat, tr_flat,
               out_betas_t, out_bp, out_go_t, out_tr_t,
               idx_all, widx_v, mc_v, mp_v, slab_a, slab_b, ext_v,
               brow_v, betas_v, go_v, tr_v,
               sem_bp, sem_sm, sem_x, sem_sl):
    wid = lax.axis_index("s") * _NUM_CORES + lax.axis_index("c")
    base = wid * _BPW
    iota16 = jnp.arange(16, dtype=jnp.int32)

    pltpu.sync_copy(frame_ids, idx_all)

    for g in range(_BPW // 16):
        vec = idx_all[pl.ds(base + g * 16, 16)]
        for r in range(_D_SMALL):
            widx_v[r, pl.ds(g * 16, 16)] = vec + (r * _NUM_FRAMES)
    for r in range(_D_SMALL):
        pltpu.make_async_copy(
            go_flat.at[widx_v.at[r]], go_v.at[r], sem_sm).start()
        pltpu.make_async_copy(
            tr_flat.at[widx_v.at[r]], tr_v.at[r], sem_sm).start()

    lo = wid * _FPW
    hi = lo + _FPW
    slab_off = pl.multiple_of((lo // _LANE) * _LANE, _LANE)

    def scan(g, off):
        vec = idx_all[pl.ds(g * 16, 16)]
        m = jnp.logical_and(vec >= lo, vec < hi)
        plsc.store_compressed(mc_v.at[pl.ds(off, 16)], vec, mask=m)
        pos = g * 16 + iota16
        plsc.store_compressed(mp_v.at[pl.ds(off, 16)], pos, mask=m)
        return off + plsc.all_reduce_population_count(m)[0]

    num_match = lax.fori_loop(0, _BATCH // 16, scan, jnp.int32(0))

    pltpu.sync_copy(betas_w, brow_v)
    brow = brow_v[0, :]
    for j in range(_D_BETAS):
        val = jnp.full((16,), brow[j], jnp.float32)
        for k in range(_BPW // 16):
            betas_v[j, pl.ds(k * 16, 16)] = val

    def chunk_body(ch, carry):
        cbase = ch * _CH
        n_valid = jnp.minimum(_CH, num_match - cbase)
        n_mg = (n_valid + 15) // 16
        slabs = (slab_a, slab_b)

        def start_slab(s):
            nrows = min(8, _D_BP - s * 8)
            buf = slabs[s % 2]

            @pl.when(wid < _NUM_WORKERS - 1)
            def _():
                pltpu.make_async_copy(
                    bp_t.at[pl.ds(s * 8, nrows), pl.ds(slab_off, _SLAB_W)],
                    buf.at[pl.ds(0, nrows)], sem_sl).start()

            @pl.when(wid == _NUM_WORKERS - 1)
            def _():
                pltpu.make_async_copy(
                    bp_t.at[pl.ds(s * 8, nrows), pl.ds(_LAST_OFF, _SLAB_W_LAST)],
                    buf.at[pl.ds(0, nrows), pl.ds(0, _SLAB_W_LAST)],
                    sem_sl).start()

        def wait_slab(s):
            nrows = min(8, _D_BP - s * 8)
            buf = slabs[s % 2]

            @pl.when(wid < _NUM_WORKERS - 1)
            def _():
                pltpu.make_async_copy(
                    bp_t.at[pl.ds(s * 8, nrows), pl.ds(slab_off, _SLAB_W)],
                    buf.at[pl.ds(0, nrows)], sem_sl).wait()

            @pl.when(wid == _NUM_WORKERS - 1)
            def _():
                pltpu.make_async_copy(
                    bp_t.at[pl.ds(s * 8, nrows), pl.ds(_LAST_OFF, _SLAB_W_LAST)],
                    buf.at[pl.ds(0, nrows), pl.ds(0, _SLAB_W_LAST)],
                    sem_sl).wait()

        start_slab(0)
        for s in range(_NSG):
            nrows = min(8, _D_BP - s * 8)
            slab_v = slabs[s % 2]
            wait_slab(s)
            if s + 1 < _NSG:
                start_slab(s + 1)

            def extract(mg, c2):
                mbase = cbase + mg * 16
                cvec = mc_v[pl.ds(mbase, 16)]
                valid = (mbase + iota16) < num_match
                sel = jnp.where(valid, cvec - slab_off, 0)
                erow = mg * 16 + iota16
                for r8 in range(nrows):
                    vals = plsc.load_gather(
                        slab_v, [jnp.full((16,), r8, jnp.int32), sel])
                    plsc.store_scatter(
                        ext_v, [erow, jnp.full((16,), s * 8 + r8, jnp.int32)],
                        vals, mask=valid)
                return c2

            lax.fori_loop(0, n_mg, extract, 0)

        def fire(mg, c2):
            pvec = mp_v[pl.ds(cbase + mg * 16, 16)]
            cvec = mc_v[pl.ds(cbase + mg * 16, 16)]
            for j in range(16):
                v = (cbase + mg * 16 + j) < num_match
                in_tail = cvec[j] >= _TAIL_START

                @pl.when(jnp.logical_and(v, jnp.logical_not(in_tail)))
                def _():
                    pltpu.make_async_copy(
                        ext_v.at[pl.ds(mg * 16 + j, 1)],
                        out_bp.at[pl.ds(pvec[j], 1)], sem_bp).start()

                @pl.when(jnp.logical_and(v, in_tail))
                def _():
                    pltpu.make_async_copy(
                        bp_tail.at[pl.ds(cvec[j] - _TAIL_START, 1)],
                        out_bp.at[pl.ds(pvec[j], 1)], sem_bp).start()
            return c2

        lax.fori_loop(0, n_mg, fire, 0)

        def drain(_, c2):
            pltpu.make_async_copy(
                ext_v.at[pl.ds(0, 1)], out_bp.at[pl.ds(0, 1)], sem_bp).wait()
            return c2

        lax.fori_loop(0, n_valid, drain, 0)
        return carry

    nch = (num_match + _CH - 1) // _CH
    lax.fori_loop(0, nch, chunk_body, 0)

    pltpu.make_async_copy(go_flat.at[pl.ds(0, _BPW)], go_v.at[0], sem_sm).wait()
    pltpu.make_async_copy(go_flat.at[pl.ds(0, _BPW)], go_v.at[1], sem_sm).wait()
    pltpu.make_async_copy(go_flat.at[pl.ds(0, _BPW)], go_v.at[2], sem_sm).wait()
    pltpu.make_async_copy(tr_flat.at[pl.ds(0, _BPW)], tr_v.at[0], sem_sm).wait()
    pltpu.make_async_copy(tr_flat.at[pl.ds(0, _BPW)], tr_v.at[1], sem_sm).wait()
    pltpu.make_async_copy(tr_flat.at[pl.ds(0, _BPW)], tr_v.at[2], sem_sm).wait()

    lbase = pl.multiple_of(base, _LANE)
    c1 = pltpu.make_async_copy(go_v, out_go_t.at[:, pl.ds(lbase, _BPW)], sem_x)
    c2 = pltpu.make_async_copy(tr_v, out_tr_t.at[:, pl.ds(lbase, _BPW)], sem_x)
    c3 = pltpu.make_async_copy(
        betas_v, out_betas_t.at[:, pl.ds(lbase, _BPW)], sem_x)
    c1.start()
    c2.start()
    c3.start()
    c1.wait()
    c2.wait()
    c3.wait()


@jax.jit
def _run(frame_ids, betas_w, body_pose_w, global_orient_w, transl_w):
    call = pl.kernel(
        _sc_lookup,
        out_type=(
            jax.ShapeDtypeStruct((_D_BETAS, _BATCH), jnp.float32),
            jax.ShapeDtypeStruct((_BATCH, _D_BP), jnp.float32),
            jax.ShapeDtypeStruct((_D_SMALL, _BATCH), jnp.float32),
            jax.ShapeDtypeStruct((_D_SMALL, _BATCH), jnp.float32),
        ),
        mesh=plsc.VectorSubcoreMesh(
            core_axis_name="c", subcore_axis_name="s",
            num_cores=_NUM_CORES, num_subcores=_NUM_SUBCORES,
        ),
        scratch_types=[
            pltpu.VMEM((_BATCH,), jnp.int32),
            pltpu.VMEM((_D_SMALL, _BPW), jnp.int32),
            pltpu.VMEM((_BATCH + 16,), jnp.int32),
            pltpu.VMEM((_BATCH + 16,), jnp.int32),
            pltpu.VMEM((8, _SLAB_W), jnp.float32),
            pltpu.VMEM((8, _SLAB_W), jnp.float32),
            pltpu.VMEM((_CH, _D_BP), jnp.float32),
            pltpu.VMEM((1, _D_BETAS), jnp.float32),
            pltpu.VMEM((_D_BETAS, _BPW), jnp.float32),
            pltpu.VMEM((_D_SMALL, _BPW), jnp.float32),
            pltpu.VMEM((_D_SMALL, _BPW), jnp.float32),
            pltpu.SemaphoreType.DMA,
            pltpu.SemaphoreType.DMA,
            pltpu.SemaphoreType.DMA,
            pltpu.SemaphoreType.DMA,
        ],
        compiler_params=pltpu.CompilerParams(
            use_tc_tiling_on_sc=True, needs_layout_passes=False),
    )
    betas_t, bp, go_res_t, tr_res_t = call(
        frame_ids, betas_w, body_pose_w.T, body_pose_w[_TAIL_START:],
        global_orient_w.T.reshape(-1), transl_w.T.reshape(-1))
    return betas_t.T, bp, go_res_t.T, tr_res_t.T


def kernel(frame_ids, betas_w, body_pose_w, global_orient_w, transl_w):
    return _run(frame_ids, betas_w, body_pose_w, global_orient_w, transl_w)
